# Initial kernel scaffold; baseline (speedup 1.0000x reference)
import jax
import jax.numpy as jnp
from jax.experimental import pallas as pl


def kernel(V, U, center_word, target_word, outsiede_words):
    raise NotImplementedError("write your pallas kernel here")


# speedup vs baseline: 5.5109x; 5.5109x over previous
import functools

import jax
import jax.numpy as jnp
from jax import lax
from jax.experimental import pallas as pl
from jax.experimental.pallas import tpu as pltpu
from jax.experimental.pallas import tpu_sc as plsc

SUB = 32
LANES = 16


def _make_sc_kernel(B, K, D, NC, NS):
    NW = NC * NS
    RW = B // NW
    NSUB = RW // SUB
    DV = D // LANES

    mesh = plsc.VectorSubcoreMesh(core_axis_name="c", subcore_axis_name="s")

    @functools.partial(
        pl.kernel,
        out_type=(
            jax.ShapeDtypeStruct((B,), jnp.float32),
            jax.ShapeDtypeStruct((B,), jnp.float32),
        ),
        mesh=mesh,
        compiler_params=pltpu.CompilerParams(
            needs_layout_passes=False, use_tc_tiling_on_sc=False),
        scratch_types=[
            pltpu.VMEM((RW,), jnp.int32),
            pltpu.VMEM((RW,), jnp.int32),
            pltpu.VMEM((NSUB * K * SUB,), jnp.int32),
            pltpu.VMEM((2, SUB, D), jnp.float32),
            pltpu.VMEM((2, SUB, D), jnp.float32),
            pltpu.VMEM((2, K * SUB, D), jnp.float32),
            pltpu.VMEM((RW,), jnp.float32),
            pltpu.VMEM((RW,), jnp.float32),
            pltpu.SemaphoreType.DMA,
            pltpu.SemaphoreType.DMA,
        ],
    )
    def sc_kernel(vt, ut, ci, ti, oi, num_out, se_out,
                  cidx, tidx, oidx, crows, trows, orows,
                  numst, sest, sem0, sem1):
        wid = lax.axis_index("s") * NC + lax.axis_index("c")
        wbase = wid * RW
        sems = (sem0, sem1)

        OPW = NSUB * K * SUB
        idx_copies = [
            (ci.at[pl.ds(wbase, RW)], cidx),
            (ti.at[pl.ds(wbase, RW)], tidx),
            (oi.at[pl.ds(wid * OPW, OPW)], oidx),
        ]
        for s_, d_ in idx_copies:
            pltpu.make_async_copy(s_, d_, sem0).start()
        for s_, d_ in idx_copies:
            pltpu.make_async_copy(s_, d_, sem0).wait()

        NG = (K * SUB) // 128

        def gather_set(j, b):
            base = j * SUB
            lst = [
                (vt.at[cidx.at[pl.ds(base, SUB)]], crows.at[b]),
                (ut.at[tidx.at[pl.ds(base, SUB)]], trows.at[b]),
            ]
            for g in range(NG):
                lst.append((ut.at[oidx.at[pl.ds(j * K * SUB + g * 128, 128)]],
                            orows.at[b, pl.ds(g * 128, 128)]))
            return lst

        def start(j, b):
            for s_, d_ in gather_set(j, b):
                pltpu.make_async_copy(s_, d_, sems[b]).start()

        def drain(j, b):
            for s_, d_ in gather_set(j, b):
                pltpu.make_async_copy(s_, d_, sems[b]).wait()

        def compute(j, b):
            base = j * SUB
            lane_ids = lax.iota(jnp.int32, LANES)

            def grp_body(h, carry):
                def lane_body(rr, acc):
                    r = h * LANES + rr
                    msk = lane_ids == rr
                    cs = [crows[b, r, pl.ds(LANES * i, LANES)]
                          for i in range(DV)]
                    ts = [trows[b, r, pl.ds(LANES * i, LANES)]
                          for i in range(DV)]
                    s = cs[0] * ts[0]
                    for i in range(1, DV):
                        s = s + cs[i] * ts[i]
                    numv = jnp.where(msk, jnp.sum(s), acc[0])
                    denv = list(acc[1:])
                    for k in range(K):
                        od = [orows[b, k * SUB + r, pl.ds(LANES * i, LANES)]
                              for i in range(DV)]
                        sd = cs[0] * od[0]
                        for i in range(1, DV):
                            sd = sd + cs[i] * od[i]
                        denv[k] = jnp.where(msk, jnp.sum(sd), denv[k])
                    return (numv, *denv)

                zeros = jnp.zeros((LANES,), jnp.float32)
                res = lax.fori_loop(0, LANES, lane_body,
                                    (zeros,) * (K + 1))
                numv, denv = res[0], res[1:]
                seacc = jnp.exp(denv[0])
                for k in range(1, K):
                    seacc = seacc + jnp.exp(denv[k])
                numst[pl.ds(base + h * LANES, LANES)] = numv
                sest[pl.ds(base + h * LANES, LANES)] = seacc
                return carry

            lax.fori_loop(0, SUB // LANES, grp_body, 0)

        start(0, 0)
        start(1, 1)

        def jj_body(jj, carry):
            for b in range(2):
                j = 2 * jj + b
                drain(j, b)
                compute(j, b)

                @pl.when(jj < NSUB // 2 - 1)
                def _():
                    start(j + 2, b)

            return carry

        lax.fori_loop(0, NSUB // 2, jj_body, 0)

        pltpu.sync_copy(numst, num_out.at[pl.ds(wbase, RW)])
        pltpu.sync_copy(sest, se_out.at[pl.ds(wbase, RW)])

    return sc_kernel


def _tc_loss(num_ref, se_ref, out_ref):
    loss = jnp.mean(jnp.log(se_ref[...])) - jnp.mean(num_ref[...])
    out_ref[...] = jnp.full((1, 1), loss, jnp.float32)


def kernel(V, U, center_word, target_word, outsiede_words):
    B, K = outsiede_words.shape
    D = V.shape[1]
    c = center_word.reshape(-1).astype(jnp.int32)
    t = target_word.reshape(-1).astype(jnp.int32)

    info = plsc.get_sparse_core_info()
    NW = info.num_cores * info.num_subcores
    RW = B // NW
    NSUB = RW // SUB
    ot = (outsiede_words.astype(jnp.int32)
          .reshape(NW, NSUB, SUB, K)
          .transpose(0, 1, 3, 2)
          .reshape(-1))

    sc = _make_sc_kernel(B, K, D, info.num_cores, info.num_subcores)
    num, se = sc(V, U, c, t, ot)

    rows = B // 128
    loss = pl.pallas_call(
        _tc_loss,
        out_shape=jax.ShapeDtypeStruct((1, 1), jnp.float32),
    )(num.reshape(rows, 128), se.reshape(rows, 128))
    return loss[0, 0]


# speedup vs baseline: 8.5131x; 1.5448x over previous
import functools

import jax
import jax.numpy as jnp
from jax import lax
from jax.experimental import pallas as pl
from jax.experimental.pallas import tpu as pltpu
from jax.experimental.pallas import tpu_sc as plsc

SUB = 32
LANES = 16


def _make_center_gather(B, D, VOCAB, NC, NS):
    NW = NC * NS
    RW = B // NW
    W = 2
    DV = D // LANES

    mesh = plsc.VectorSubcoreMesh(core_axis_name="c", subcore_axis_name="s")

    @functools.partial(
        pl.kernel,
        out_type=jax.ShapeDtypeStruct((B, D), jnp.float32),
        mesh=mesh,
        compiler_params=pltpu.CompilerParams(
            needs_layout_passes=False, disable_bounds_checks=True),
        scratch_types=[
            pltpu.VMEM((RW,), jnp.int32),
            pltpu.VMEM((2, W, D, 128), jnp.float32),
            pltpu.VMEM((RW, D), jnp.float32),
            pltpu.SemaphoreType.DMA,
            pltpu.SemaphoreType.DMA,
        ],
    )
    def cg_kernel(vtt, ci, out, cidx_v, tbuf, rbuf, sem0, sem1):
        wid = lax.axis_index("s") * NC + lax.axis_index("c")
        wbase = wid * RW
        sems = (sem0, sem1)
        pltpu.sync_copy(ci.at[pl.ds(wbase, RW)], cidx_v)
        lane_ids = lax.iota(jnp.int32, LANES)

        def grp_body(g, carry):
            idx16 = cidx_v[pl.ds(g * LANES, LANES)]
            iscal = [jnp.sum(jnp.where(lane_ids == q, idx16, 0))
                     for q in range(LANES)]

            def dma(p, q):
                i = iscal[2 * p + q]
                off = pl.multiple_of((i >> 7) * 128, 128)
                return pltpu.make_async_copy(
                    vtt.at[:, pl.ds(off, 128)], tbuf.at[p % 2, q],
                    sems[p % 2])

            def issue(p):
                for q in range(W):
                    dma(p, q).start()

            issue(0)
            for p in range(LANES // W):
                if p + 1 < LANES // W:
                    issue(p + 1)
                for q in range(W):
                    dma(p, q).wait()
                    i = iscal[2 * p + q]
                    col = jnp.full((LANES,), i & 127, jnp.int32)
                    for qq in range(DV):
                        rows16 = lane_ids + LANES * qq
                        v = plsc.load_gather(tbuf.at[p % 2, q],
                                             [rows16, col])
                        rbuf[g * LANES + 2 * p + q,
                             pl.ds(LANES * qq, LANES)] = v
            return carry

        lax.fori_loop(0, RW // LANES, grp_body, 0)
        pltpu.sync_copy(rbuf, out.at[pl.ds(wbase, RW)])

    return cg_kernel


def _make_sc_kernel(B, K, D, NC, NS):
    NW = NC * NS
    RW = B // NW
    NSUB = RW // SUB
    DV = D // LANES

    mesh = plsc.VectorSubcoreMesh(core_axis_name="c", subcore_axis_name="s")

    @functools.partial(
        pl.kernel,
        out_type=(
            jax.ShapeDtypeStruct((B,), jnp.float32),
            jax.ShapeDtypeStruct((B,), jnp.float32),
        ),
        mesh=mesh,
        compiler_params=pltpu.CompilerParams(
            needs_layout_passes=False, use_tc_tiling_on_sc=False),
        scratch_types=[
            pltpu.VMEM((RW,), jnp.int32),
            pltpu.VMEM((NSUB * K * SUB,), jnp.int32),
            pltpu.VMEM((2, SUB, D), jnp.float32),
            pltpu.VMEM((2, SUB, D), jnp.float32),
            pltpu.VMEM((2, K * SUB, D), jnp.float32),
            pltpu.VMEM((RW,), jnp.float32),
            pltpu.VMEM((RW,), jnp.float32),
            pltpu.SemaphoreType.DMA,
            pltpu.SemaphoreType.DMA,
        ],
    )
    def sc_kernel(cg, ut, ti, oi, num_out, se_out,
                  tidx, oidx, crows, trows, orows,
                  numst, sest, sem0, sem1):
        wid = lax.axis_index("s") * NC + lax.axis_index("c")
        wbase = wid * RW
        sems = (sem0, sem1)

        OPW = NSUB * K * SUB
        idx_copies = [
            (ti.at[pl.ds(wbase, RW)], tidx),
            (oi.at[pl.ds(wid * OPW, OPW)], oidx),
        ]
        for s_, d_ in idx_copies:
            pltpu.make_async_copy(s_, d_, sem0).start()
        for s_, d_ in idx_copies:
            pltpu.make_async_copy(s_, d_, sem0).wait()

        NG = (K * SUB) // 128

        def gather_set(j, b):
            base = j * SUB
            lst = [
                (cg.at[pl.ds(wbase + base, SUB)], crows.at[b]),
                (ut.at[tidx.at[pl.ds(base, SUB)]], trows.at[b]),
            ]
            for g in range(NG):
                lst.append((ut.at[oidx.at[pl.ds(j * K * SUB + g * 128, 128)]],
                            orows.at[b, pl.ds(g * 128, 128)]))
            return lst

        def start(j, b):
            for s_, d_ in gather_set(j, b):
                pltpu.make_async_copy(s_, d_, sems[b]).start()

        def drain(j, b):
            for s_, d_ in gather_set(j, b):
                pltpu.make_async_copy(s_, d_, sems[b]).wait()

        def compute(j, b):
            base = j * SUB
            lane_ids = lax.iota(jnp.int32, LANES)

            def grp_body(h, carry):
                def lane_body(rr, acc):
                    r = h * LANES + rr
                    msk = lane_ids == rr
                    cs = [crows[b, r, pl.ds(LANES * i, LANES)]
                          for i in range(DV)]
                    ts = [trows[b, r, pl.ds(LANES * i, LANES)]
                          for i in range(DV)]
                    s = cs[0] * ts[0]
                    for i in range(1, DV):
                        s = s + cs[i] * ts[i]
                    numv = jnp.where(msk, jnp.sum(s), acc[0])
                    denv = list(acc[1:])
                    for k in range(K):
                        od = [orows[b, k * SUB + r, pl.ds(LANES * i, LANES)]
                              for i in range(DV)]
                        sd = cs[0] * od[0]
                        for i in range(1, DV):
                            sd = sd + cs[i] * od[i]
                        denv[k] = jnp.where(msk, jnp.sum(sd), denv[k])
                    return (numv, *denv)

                zeros = jnp.zeros((LANES,), jnp.float32)
                res = lax.fori_loop(0, LANES, lane_body,
                                    (zeros,) * (K + 1))
                numv, denv = res[0], res[1:]
                seacc = jnp.exp(denv[0])
                for k in range(1, K):
                    seacc = seacc + jnp.exp(denv[k])
                numst[pl.ds(base + h * LANES, LANES)] = numv
                sest[pl.ds(base + h * LANES, LANES)] = seacc
                return carry

            lax.fori_loop(0, SUB // LANES, grp_body, 0)

        start(0, 0)
        start(1, 1)

        def jj_body(jj, carry):
            for b in range(2):
                j = 2 * jj + b
                drain(j, b)
                compute(j, b)

                @pl.when(jj < NSUB // 2 - 1)
                def _():
                    start(j + 2, b)

            return carry

        lax.fori_loop(0, NSUB // 2, jj_body, 0)

        pltpu.sync_copy(numst, num_out.at[pl.ds(wbase, RW)])
        pltpu.sync_copy(sest, se_out.at[pl.ds(wbase, RW)])

    return sc_kernel


def _tc_loss(num_ref, se_ref, out_ref):
    loss = jnp.mean(jnp.log(se_ref[...])) - jnp.mean(num_ref[...])
    out_ref[...] = jnp.full((1, 1), loss, jnp.float32)


def kernel(V, U, center_word, target_word, outsiede_words):
    B, K = outsiede_words.shape
    D = V.shape[1]
    c = center_word.reshape(-1).astype(jnp.int32)
    t = target_word.reshape(-1).astype(jnp.int32)

    info = plsc.get_sparse_core_info()
    NW = info.num_cores * info.num_subcores
    RW = B // NW
    NSUB = RW // SUB
    ot = (outsiede_words.astype(jnp.int32)
          .reshape(NW, NSUB, SUB, K)
          .transpose(0, 1, 3, 2)
          .reshape(-1))

    cgk = _make_center_gather(B, D, V.shape[0], info.num_cores,
                              info.num_subcores)
    cg = cgk(V.T, c)

    sc = _make_sc_kernel(B, K, D, info.num_cores, info.num_subcores)
    num, se = sc(cg, U, t, ot)

    rows = B // 128
    loss = pl.pallas_call(
        _tc_loss,
        out_shape=jax.ShapeDtypeStruct((1, 1), jnp.float32),
    )(num.reshape(rows, 128), se.reshape(rows, 128))
    return loss[0, 0]
